# baseline (device time: 67589 ns/iter reference)
import jax
import jax.numpy as jnp
from jax import lax
from jax.experimental import pallas as pl
from jax.experimental.pallas import tpu as pltpu

N_DEV = 4
SQ = 512
D = 1024
H = 8
DH = 128
SKV = 2048
SCALE = 0.08838834764831843

NB = 4
BR = SQ // NB
NI = NB * H


def kernel(x, Wq, Wo, K_ext, V_ext):
    def body(x_ref, wq_ref, wo_ref, k_hbm, v_hbm, out_ref,
             k_buf, v_buf, acc_ref, r1_ref, r2_ref,
             kv_sems, ssems, rsems):
        my = lax.axis_index("i")
        p1 = my ^ 1
        p2 = 3 - my

        barrier = pltpu.get_barrier_semaphore()
        for nbr in (p1, p2):
            pl.semaphore_signal(barrier, inc=1, device_id=(nbr,),
                                device_id_type=pl.DeviceIdType.MESH)
        pl.semaphore_wait(barrier, 2)

        def kv_copies(i):
            h = i % H
            slot = i % 2
            return (
                pltpu.make_async_copy(
                    k_hbm.at[0, :, h, :], k_buf.at[slot], kv_sems.at[slot, 0]),
                pltpu.make_async_copy(
                    v_hbm.at[0, :, h, :], v_buf.at[slot], kv_sems.at[slot, 1]),
            )

        def compute_block(b):
            for h in range(H):
                i = b * H + h
                if i + 1 < NI:
                    for c in kv_copies(i + 1):
                        c.start()
                for c in kv_copies(i):
                    c.wait()
                slot = i % 2
                xb = x_ref[0, b * BR:(b + 1) * BR, :]
                q = jnp.dot(xb, wq_ref[:, h * DH:(h + 1) * DH],
                            preferred_element_type=jnp.float32)
                s = lax.dot_general(
                    q, k_buf[slot], (((1,), (1,)), ((), ())),
                    preferred_element_type=jnp.float32) * SCALE
                m = jnp.max(s, axis=1, keepdims=True)
                p = jnp.exp(s - m)
                l = jnp.sum(p, axis=1, keepdims=True)
                o = jnp.dot(p, v_buf[slot],
                            preferred_element_type=jnp.float32) / l
                part = jnp.dot(o, wo_ref[h * DH:(h + 1) * DH, :],
                               preferred_element_type=jnp.float32)
                if h == 0:
                    acc_ref[b] = part
                else:
                    acc_ref[b] = acc_ref[b] + part

        def ex(b, stage, partner, rbuf):
            return pltpu.make_async_remote_copy(
                src_ref=acc_ref.at[b],
                dst_ref=rbuf.at[b],
                send_sem=ssems.at[b, stage],
                recv_sem=rsems.at[b, stage],
                device_id=(partner,),
                device_id_type=pl.DeviceIdType.MESH,
            )

        ex1 = [ex(b, 0, p1, r1_ref) for b in range(NB)]
        ex2 = [ex(b, 1, p2, r2_ref) for b in range(NB)]

        def s1_done(b):
            ex1[b].wait()
            acc_ref[b] = acc_ref[b] + r1_ref[b]
            ex2[b].start()

        def s2_done(b):
            ex2[b].wait()
            out_ref[0, b * BR:(b + 1) * BR, :] = acc_ref[b] + r2_ref[b]

        for c in kv_copies(0):
            c.start()

        compute_block(0)
        ex1[0].start()
        compute_block(1)
        s1_done(0)
        ex1[1].start()
        compute_block(2)
        s2_done(0)
        s1_done(1)
        ex1[2].start()
        compute_block(3)
        s2_done(1)
        s1_done(2)
        ex1[3].start()
        s2_done(2)
        s1_done(3)
        s2_done(3)

    return pl.pallas_call(
        body,
        out_shape=jax.ShapeDtypeStruct((1, SQ, D), jnp.float32),
        in_specs=[
            pl.BlockSpec(memory_space=pltpu.VMEM),
            pl.BlockSpec(memory_space=pltpu.VMEM),
            pl.BlockSpec(memory_space=pltpu.VMEM),
            pl.BlockSpec(memory_space=pl.ANY),
            pl.BlockSpec(memory_space=pl.ANY),
        ],
        out_specs=pl.BlockSpec(memory_space=pltpu.VMEM),
        scratch_shapes=[
            pltpu.VMEM((2, SKV, DH), jnp.float32),
            pltpu.VMEM((2, SKV, DH), jnp.float32),
            pltpu.VMEM((NB, BR, D), jnp.float32),
            pltpu.VMEM((NB, BR, D), jnp.float32),
            pltpu.VMEM((NB, BR, D), jnp.float32),
            pltpu.SemaphoreType.DMA((2, 2)),
            pltpu.SemaphoreType.DMA((NB, 2)),
            pltpu.SemaphoreType.DMA((NB, 2)),
        ],
        compiler_params=pltpu.CompilerParams(
            collective_id=0,
            vmem_limit_bytes=100 * 1024 * 1024,
        ),
    )(x, Wq, Wo, K_ext, V_ext)


# device time: 63994 ns/iter; 1.0562x vs baseline; 1.0562x over previous
import jax
import jax.numpy as jnp
from jax import lax
from jax.experimental import pallas as pl
from jax.experimental.pallas import tpu as pltpu

N_DEV = 4
SQ = 512
D = 1024
H = 8
DH = 128
SKV = 2048
SCALE = 0.08838834764831843

NB = 4
BR = SQ // NB


def kernel(x, Wq, Wo, K_ext, V_ext):
    def body(x_ref, wq_ref, wo_ref, k_hbm, v_hbm, out_ref,
             k_buf, v_buf, kbf_ref, vbf_ref, xbf_ref, wqbf_ref, wobf_ref,
             acc_ref, r1_ref, r2_ref,
             kv_sems, ssems, rsems):
        my = lax.axis_index("i")
        p1 = my ^ 1
        p2 = 3 - my

        barrier = pltpu.get_barrier_semaphore()
        for nbr in (p1, p2):
            pl.semaphore_signal(barrier, inc=1, device_id=(nbr,),
                                device_id_type=pl.DeviceIdType.MESH)
        pl.semaphore_wait(barrier, 2)

        def kv_copies(h):
            slot = h % 2
            return (
                pltpu.make_async_copy(
                    k_hbm.at[0, :, h, :], k_buf.at[slot], kv_sems.at[slot, 0]),
                pltpu.make_async_copy(
                    v_hbm.at[0, :, h, :], v_buf.at[slot], kv_sems.at[slot, 1]),
            )

        for c in kv_copies(0):
            c.start()

        xbf_ref[...] = x_ref[0].astype(jnp.bfloat16)
        wqbf_ref[...] = wq_ref[...].astype(jnp.bfloat16)
        wobf_ref[...] = wo_ref[...].astype(jnp.bfloat16)

        def compute_block(b):
            for h in range(H):
                if b == 0:
                    if h + 1 < H:
                        for c in kv_copies(h + 1):
                            c.start()
                    for c in kv_copies(h):
                        c.wait()
                    slot = h % 2
                    kb = k_buf[slot].astype(jnp.bfloat16)
                    vb = v_buf[slot].astype(jnp.bfloat16)
                    kbf_ref[h] = kb
                    vbf_ref[h] = vb
                else:
                    kb = kbf_ref[h]
                    vb = vbf_ref[h]
                xb = xbf_ref[b * BR:(b + 1) * BR, :]
                q = jnp.dot(xb, wqbf_ref[:, h * DH:(h + 1) * DH],
                            preferred_element_type=jnp.float32)
                s = lax.dot_general(
                    q.astype(jnp.bfloat16), kb, (((1,), (1,)), ((), ())),
                    preferred_element_type=jnp.float32) * SCALE
                m = jnp.max(s, axis=1, keepdims=True)
                p = jnp.exp(s - m)
                l = jnp.sum(p, axis=1, keepdims=True)
                o = jnp.dot(p.astype(jnp.bfloat16), vb,
                            preferred_element_type=jnp.float32) / l
                part = jnp.dot(o.astype(jnp.bfloat16),
                               wobf_ref[h * DH:(h + 1) * DH, :],
                               preferred_element_type=jnp.float32)
                if h == 0:
                    acc_ref[b] = part
                else:
                    acc_ref[b] = acc_ref[b] + part

        def ex(b, stage, partner, rbuf):
            return pltpu.make_async_remote_copy(
                src_ref=acc_ref.at[b],
                dst_ref=rbuf.at[b],
                send_sem=ssems.at[b, stage],
                recv_sem=rsems.at[b, stage],
                device_id=(partner,),
                device_id_type=pl.DeviceIdType.MESH,
            )

        ex1 = [ex(b, 0, p1, r1_ref) for b in range(NB)]
        ex2 = [ex(b, 1, p2, r2_ref) for b in range(NB)]

        def s1_done(b):
            ex1[b].wait()
            acc_ref[b] = acc_ref[b] + r1_ref[b]
            ex2[b].start()

        def s2_done(b):
            ex2[b].wait()
            out_ref[0, b * BR:(b + 1) * BR, :] = acc_ref[b] + r2_ref[b]

        compute_block(0)
        ex1[0].start()
        compute_block(1)
        s1_done(0)
        ex1[1].start()
        compute_block(2)
        s2_done(0)
        s1_done(1)
        ex1[2].start()
        compute_block(3)
        s2_done(1)
        s1_done(2)
        ex1[3].start()
        s2_done(2)
        s1_done(3)
        s2_done(3)

    return pl.pallas_call(
        body,
        out_shape=jax.ShapeDtypeStruct((1, SQ, D), jnp.float32),
        in_specs=[
            pl.BlockSpec(memory_space=pltpu.VMEM),
            pl.BlockSpec(memory_space=pltpu.VMEM),
            pl.BlockSpec(memory_space=pltpu.VMEM),
            pl.BlockSpec(memory_space=pl.ANY),
            pl.BlockSpec(memory_space=pl.ANY),
        ],
        out_specs=pl.BlockSpec(memory_space=pltpu.VMEM),
        scratch_shapes=[
            pltpu.VMEM((2, SKV, DH), jnp.float32),
            pltpu.VMEM((2, SKV, DH), jnp.float32),
            pltpu.VMEM((H, SKV, DH), jnp.bfloat16),
            pltpu.VMEM((H, SKV, DH), jnp.bfloat16),
            pltpu.VMEM((SQ, D), jnp.bfloat16),
            pltpu.VMEM((D, D), jnp.bfloat16),
            pltpu.VMEM((D, D), jnp.bfloat16),
            pltpu.VMEM((NB, BR, D), jnp.float32),
            pltpu.VMEM((NB, BR, D), jnp.float32),
            pltpu.VMEM((NB, BR, D), jnp.float32),
            pltpu.SemaphoreType.DMA((2, 2)),
            pltpu.SemaphoreType.DMA((NB, 2)),
            pltpu.SemaphoreType.DMA((NB, 2)),
        ],
        compiler_params=pltpu.CompilerParams(
            collective_id=0,
            vmem_limit_bytes=100 * 1024 * 1024,
        ),
    )(x, Wq, Wo, K_ext, V_ext)


# device time: 46059 ns/iter; 1.4674x vs baseline; 1.3894x over previous
import jax
import jax.numpy as jnp
from jax import lax
from jax.experimental import pallas as pl
from jax.experimental.pallas import tpu as pltpu

N_DEV = 4
SQ = 512
D = 1024
H = 8
DH = 128
SKV = 2048
SCALE = 0.08838834764831843
LOG2E = 1.4426950408889634
QSCALE = SCALE * LOG2E

NB = 4
BR = SQ // NB


def kernel(x, Wq, Wo, K_ext, V_ext):
    def body(x_ref, wq_ref, wo_ref, k_hbm, v_hbm, out_ref,
             k_buf, v_buf, kbf_ref, vbf_ref, xbf_ref, wqbf_ref, wobf_ref,
             acc_ref, s1bf_ref, s2bf_ref, r1_ref, r2_ref,
             kv_sems, ssems, rsems):
        my = lax.axis_index("i")
        p1 = my ^ 1
        p2 = 3 - my

        def kv_copies(h):
            slot = h % 2
            return (
                pltpu.make_async_copy(
                    k_hbm.at[0, :, h, :], k_buf.at[slot], kv_sems.at[slot, 0]),
                pltpu.make_async_copy(
                    v_hbm.at[0, :, h, :], v_buf.at[slot], kv_sems.at[slot, 1]),
            )

        for c in kv_copies(0):
            c.start()

        xbf_ref[...] = x_ref[0].astype(jnp.bfloat16)
        wqbf_ref[...] = wq_ref[...].astype(jnp.bfloat16)
        wobf_ref[...] = wo_ref[...].astype(jnp.bfloat16)

        qa_full = jnp.dot(xbf_ref[...], wqbf_ref[...],
                          preferred_element_type=jnp.float32)

        def compute_block(b):
            qa = qa_full[b * BR:(b + 1) * BR, :]
            outs = []
            for h in range(H):
                if b == 0:
                    if h + 1 < H:
                        for c in kv_copies(h + 1):
                            c.start()
                    for c in kv_copies(h):
                        c.wait()
                    slot = h % 2
                    kb = k_buf[slot].astype(jnp.bfloat16)
                    vb = v_buf[slot].astype(jnp.bfloat16)
                    kbf_ref[h] = kb
                    vbf_ref[h] = vb
                else:
                    kb = kbf_ref[h]
                    vb = vbf_ref[h]
                qh = (qa[:, h * DH:(h + 1) * DH] * QSCALE).astype(jnp.bfloat16)
                s = lax.dot_general(
                    qh, kb, (((1,), (1,)), ((), ())),
                    preferred_element_type=jnp.float32)
                p = jnp.exp2(s)
                l = jnp.sum(p, axis=1, keepdims=True)
                o = jnp.dot(p.astype(jnp.bfloat16), vb,
                            preferred_element_type=jnp.float32) / l
                outs.append(o.astype(jnp.bfloat16))
            attn = jnp.concatenate(outs, axis=1)
            pr = jnp.dot(attn, wobf_ref[...],
                         preferred_element_type=jnp.float32)
            acc_ref[b] = pr
            s1bf_ref[b] = pr.astype(jnp.bfloat16)

        def ex(b, stage, partner, sbuf, rbuf):
            return pltpu.make_async_remote_copy(
                src_ref=sbuf.at[b],
                dst_ref=rbuf.at[b],
                send_sem=ssems.at[b, stage],
                recv_sem=rsems.at[b, stage],
                device_id=(partner,),
                device_id_type=pl.DeviceIdType.MESH,
            )

        ex1 = [ex(b, 0, p1, s1bf_ref, r1_ref) for b in range(NB)]
        ex2 = [ex(b, 1, p2, s2bf_ref, r2_ref) for b in range(NB)]

        def s1_done(b):
            ex1[b].wait()
            a = acc_ref[b] + r1_ref[b].astype(jnp.float32)
            acc_ref[b] = a
            s2bf_ref[b] = a.astype(jnp.bfloat16)
            ex2[b].start()

        def s2_done(b):
            ex2[b].wait()
            out_ref[0, b * BR:(b + 1) * BR, :] = (
                acc_ref[b] + r2_ref[b].astype(jnp.float32))

        compute_block(0)

        barrier = pltpu.get_barrier_semaphore()
        for nbr in (p1, p2):
            pl.semaphore_signal(barrier, inc=1, device_id=(nbr,),
                                device_id_type=pl.DeviceIdType.MESH)
        pl.semaphore_wait(barrier, 2)

        ex1[0].start()
        for b in range(1, NB):
            compute_block(b)
            if b >= 2:
                s2_done(b - 2)
            s1_done(b - 1)
            ex1[b].start()
        s2_done(NB - 2)
        s1_done(NB - 1)
        s2_done(NB - 1)

    return pl.pallas_call(
        body,
        out_shape=jax.ShapeDtypeStruct((1, SQ, D), jnp.float32),
        in_specs=[
            pl.BlockSpec(memory_space=pltpu.VMEM),
            pl.BlockSpec(memory_space=pltpu.VMEM),
            pl.BlockSpec(memory_space=pltpu.VMEM),
            pl.BlockSpec(memory_space=pl.ANY),
            pl.BlockSpec(memory_space=pl.ANY),
        ],
        out_specs=pl.BlockSpec(memory_space=pltpu.VMEM),
        scratch_shapes=[
            pltpu.VMEM((2, SKV, DH), jnp.float32),
            pltpu.VMEM((2, SKV, DH), jnp.float32),
            pltpu.VMEM((H, SKV, DH), jnp.bfloat16),
            pltpu.VMEM((H, SKV, DH), jnp.bfloat16),
            pltpu.VMEM((SQ, D), jnp.bfloat16),
            pltpu.VMEM((D, D), jnp.bfloat16),
            pltpu.VMEM((D, D), jnp.bfloat16),
            pltpu.VMEM((NB, BR, D), jnp.float32),
            pltpu.VMEM((NB, BR, D), jnp.bfloat16),
            pltpu.VMEM((NB, BR, D), jnp.bfloat16),
            pltpu.VMEM((NB, BR, D), jnp.bfloat16),
            pltpu.VMEM((NB, BR, D), jnp.bfloat16),
            pltpu.SemaphoreType.DMA((2, 2)),
            pltpu.SemaphoreType.DMA((NB, 2)),
            pltpu.SemaphoreType.DMA((NB, 2)),
        ],
        compiler_params=pltpu.CompilerParams(
            collective_id=0,
            vmem_limit_bytes=100 * 1024 * 1024,
        ),
    )(x, Wq, Wo, K_ext, V_ext)
